# baseline (speedup 1.0000x reference)
import jax
import jax.numpy as jnp
from jax import lax
from jax.experimental import pallas as pl
from jax.experimental.pallas import tpu as pltpu
from jax.experimental.pallas import tpu_sc as plsc

_BATCH = 16384
_EMB = 32
_NC = 2
_NS = 16
_NW = _NC * _NS
_BPW = _BATCH // _NW
_CHUNK = 128
_NCHUNK = _BPW // _CHUNK
_LANES = 16


def _gmf_body(u_hbm, v_hbm, ut_hbm, vt_hbm, out_hbm,
              u_idx, v_idx, u_rows, v_rows, o_rows, sem):
    wid = lax.axis_index("s") * _NC + lax.axis_index("c")
    base = wid * _BPW
    pltpu.sync_copy(u_hbm.at[pl.ds(base, _BPW)], u_idx)
    pltpu.sync_copy(v_hbm.at[pl.ds(base, _BPW)], v_idx)
    copies = []
    for j in range(_NCHUNK):
        sl = pl.ds(j * _CHUNK, _CHUNK)
        copies.append(pltpu.async_copy(ut_hbm.at[u_idx.at[sl]], u_rows.at[sl], sem))
        copies.append(pltpu.async_copy(vt_hbm.at[v_idx.at[sl]], v_rows.at[sl], sem))
    for c in copies:
        c.wait()

    def mul_row(r, carry):
        for c in range(0, _EMB, _LANES):
            cs = pl.ds(c, _LANES)
            o_rows[r, cs] = u_rows[r, cs] * v_rows[r, cs]
        return carry

    lax.fori_loop(0, _BPW, mul_row, 0)
    pltpu.sync_copy(o_rows, out_hbm.at[pl.ds(base, _BPW)])


@jax.jit
def _gmf(u, v, u_table, v_table):
    run = pl.kernel(
        _gmf_body,
        out_type=jax.ShapeDtypeStruct((_BATCH, _EMB), jnp.float32),
        mesh=plsc.VectorSubcoreMesh(core_axis_name="c", subcore_axis_name="s"),
        scratch_types=[
            pltpu.VMEM((_BPW,), jnp.int32),
            pltpu.VMEM((_BPW,), jnp.int32),
            pltpu.VMEM((_BPW, _EMB), jnp.float32),
            pltpu.VMEM((_BPW, _EMB), jnp.float32),
            pltpu.VMEM((_BPW, _EMB), jnp.float32),
            pltpu.SemaphoreType.DMA,
        ],
        compiler_params=pltpu.CompilerParams(use_tc_tiling_on_sc=False),
    )
    return run(u, v, u_table, v_table)


def kernel(u, v, u_table, v_table):
    return _gmf(u.astype(jnp.int32), v.astype(jnp.int32), u_table, v_table)


# speedup vs baseline: 3.2596x; 3.2596x over previous
import jax
import jax.numpy as jnp
from jax import lax
from jax.experimental import pallas as pl
from jax.experimental.pallas import tpu as pltpu
from jax.experimental.pallas import tpu_sc as plsc

_BATCH = 16384
_EMB = 32
_NC = 2
_NS = 16
_NW = _NC * _NS
_BPW = _BATCH // _NW
_C = 16
_NCH = _BPW // _C


def _gmf_body(u_hbm, v_hbm, utt, vtt, out_t, ui, vi, buf, o, sem):
    wid = lax.axis_index("s") * _NC + lax.axis_index("c")
    base = wid * _BPW
    pltpu.sync_copy(u_hbm.at[pl.ds(base, _BPW)], ui)
    pltpu.sync_copy(v_hbm.at[pl.ds(base, _BPW)], vi)
    iota = lax.iota(jnp.int32, 16)
    iota_hi = iota + 16

    def chunk(ci, carry):
        rbase = ci * _C
        iu = ui[pl.ds(rbase, _C)]
        offu = (iu // 128) * 128
        ucopies = [
            pltpu.async_copy(
                utt.at[:, pl.ds(pl.multiple_of(offu[k], 128), 128)],
                buf.at[k], sem)
            for k in range(_C)
        ]
        for cp in ucopies:
            cp.wait()
        lu = iu % 128
        for k in range(_C):
            lane = jnp.full((16,), lu[k], jnp.int32)
            col = jnp.full((16,), rbase + k, jnp.int32)
            a_lo = plsc.load_gather(buf.at[k], [iota, lane])
            a_hi = plsc.load_gather(buf.at[k], [iota_hi, lane])
            plsc.store_scatter(o, [iota, col], a_lo)
            plsc.store_scatter(o, [iota_hi, col], a_hi)
        iv = vi[pl.ds(rbase, _C)]
        offv = (iv // 128) * 128
        vcopies = [
            pltpu.async_copy(
                vtt.at[:, pl.ds(pl.multiple_of(offv[k], 128), 128)],
                buf.at[k], sem)
            for k in range(_C)
        ]
        for cp in vcopies:
            cp.wait()
        lv = iv % 128
        for k in range(_C):
            lane = jnp.full((16,), lv[k], jnp.int32)
            col = jnp.full((16,), rbase + k, jnp.int32)
            b_lo = plsc.load_gather(buf.at[k], [iota, lane])
            b_hi = plsc.load_gather(buf.at[k], [iota_hi, lane])
            a_lo = plsc.load_gather(o, [iota, col])
            a_hi = plsc.load_gather(o, [iota_hi, col])
            plsc.store_scatter(o, [iota, col], a_lo * b_lo)
            plsc.store_scatter(o, [iota_hi, col], a_hi * b_hi)
        return carry

    lax.fori_loop(0, _NCH, chunk, 0)
    pltpu.sync_copy(o, out_t.at[:, pl.ds(base, _BPW)])


@jax.jit
def _gmf(u, v, utt, vtt):
    run = pl.kernel(
        _gmf_body,
        out_type=jax.ShapeDtypeStruct((_EMB, _BATCH), jnp.float32),
        mesh=plsc.VectorSubcoreMesh(core_axis_name="c", subcore_axis_name="s"),
        compiler_params=pltpu.CompilerParams(needs_layout_passes=False),
        scratch_types=[
            pltpu.VMEM((_BPW,), jnp.int32),
            pltpu.VMEM((_BPW,), jnp.int32),
            pltpu.VMEM((_C, _EMB, 128), jnp.float32),
            pltpu.VMEM((_EMB, _BPW), jnp.float32),
            pltpu.SemaphoreType.DMA,
        ],
    )
    return run(u, v, utt, vtt)


def kernel(u, v, u_table, v_table):
    out_t = _gmf(u.astype(jnp.int32), v.astype(jnp.int32),
                 u_table.T, v_table.T)
    return out_t.T
